# baseline (speedup 1.0000x reference)
import jax
import jax.numpy as jnp
from jax import lax
from jax.experimental import pallas as pl
from jax.experimental.pallas import tpu as pltpu
from jax.experimental.pallas import tpu_sc as plsc

_B = 16384
_L = 200
_VOCAB = 1000000
_DIM = 32

_NC = 2
_NS = 16
_LANES = 16
_NW = _NC * _NS
_GRP = _B // _LANES
_GPW = _GRP // _NW
_TOK = _L * _LANES
_CH = 128
_NCH = _TOK // _CH

_VB = 4096
_PGRID = -(-_VOCAB // _VB)
_PLEN = _PGRID * _VB


def _proj_body(w_ref, tbl_ref, out_ref):
    out_ref[...] = jnp.sum(tbl_ref[...] * w_ref[...], axis=1)


def _project(embd_weight, linear_weight):
    return pl.pallas_call(
        _proj_body,
        grid=(_PGRID,),
        in_specs=[
            pl.BlockSpec((1, _DIM), lambda i: (0, 0)),
            pl.BlockSpec((_VB, _DIM), lambda i: (i, 0)),
        ],
        out_specs=pl.BlockSpec((_VB,), lambda i: (i,)),
        out_shape=jax.ShapeDtypeStruct((_PLEN,), jnp.float32),
    )(linear_weight, embd_weight)


def _pool_body(p_hbm, ids_hbm, att_hbm, bias_hbm, tidx_hbm, out_hbm,
               tidx_v, gidx_v, idxt_v, attt_v, val_v,
               bias_v, out_v, ish, ash, semt, semg):
    cid = lax.axis_index("c")
    sid = lax.axis_index("s")
    wid = sid * _NC + cid
    pltpu.sync_copy(bias_hbm, bias_v)
    pltpu.sync_copy(tidx_hbm, tidx_v)
    soff = (sid * _TOK).astype(jnp.int32)
    for k in range(_L):
        ds = pl.ds(k * _LANES, _LANES)
        gidx_v[ds] = tidx_v[ds] + soff

    def group_body(gl, carry):
        g = wid * _GPW + gl
        my_ish = ish.at[pl.ds(soff, _TOK)]
        my_ash = ash.at[pl.ds(soff, _TOK)]
        pltpu.sync_copy(ids_hbm.at[pl.ds(g * _TOK, _TOK)], my_ish)
        pltpu.sync_copy(att_hbm.at[pl.ds(g * _TOK, _TOK)], my_ash)
        tcopies = []
        for j in range(_NCH):
            ds = pl.ds(j * _CH, _CH)
            tcopies.append(pltpu.async_copy(
                ish.at[gidx_v.at[ds]], idxt_v.at[ds], semt))
            tcopies.append(pltpu.async_copy(
                ash.at[gidx_v.at[ds]], attt_v.at[ds], semt))
        for c in tcopies:
            c.wait()
        gcopies = [
            pltpu.async_copy(
                p_hbm.at[idxt_v.at[pl.ds(j * _CH, _CH)]],
                val_v.at[pl.ds(j * _CH, _CH)],
                semg,
            )
            for j in range(_NCH)
        ]
        for c in gcopies:
            c.wait()

        def tok_body(i, tc):
            acc, den = tc
            a = attt_v[pl.ds(i * _LANES, _LANES)]
            v = val_v[pl.ds(i * _LANES, _LANES)]
            return acc + v * a, den + a

        zero = jnp.zeros((_LANES,), jnp.float32)
        acc, den = lax.fori_loop(0, _L, tok_body, (zero, zero))
        logit = acc / den + bias_v[...]
        out_v[...] = 1.0 / (1.0 + jnp.exp(-logit))
        pltpu.sync_copy(out_v, out_hbm.at[pl.ds(g * _LANES, _LANES)])
        return carry

    lax.fori_loop(0, _GPW, group_body, 0)


def _pool(p, ids_g, att_g, bias16, tidx):
    mesh = plsc.VectorSubcoreMesh(
        core_axis_name="c", subcore_axis_name="s",
        num_cores=_NC, num_subcores=_NS,
    )
    return pl.kernel(
        _pool_body,
        out_type=jax.ShapeDtypeStruct((_B,), jnp.float32),
        mesh=mesh,
        scratch_types=[
            pltpu.VMEM((_TOK,), jnp.int32),
            pltpu.VMEM((_TOK,), jnp.int32),
            pltpu.VMEM((_TOK,), jnp.int32),
            pltpu.VMEM((_TOK,), jnp.float32),
            pltpu.VMEM((_TOK,), jnp.float32),
            pltpu.VMEM((_LANES,), jnp.float32),
            pltpu.VMEM((_LANES,), jnp.float32),
            pltpu.VMEM_SHARED((_NS * _TOK,), jnp.int32),
            pltpu.VMEM_SHARED((_NS * _TOK,), jnp.float32),
            pltpu.SemaphoreType.DMA,
            pltpu.SemaphoreType.DMA,
        ],
    )(p, ids_g, att_g, bias16, tidx)


def kernel(ids, att_ids, embd_weight, linear_weight, linear_bias):
    ids_g = ids.astype(jnp.int32).reshape(-1)
    att_g = att_ids.astype(jnp.float32).reshape(-1)
    bias16 = jnp.broadcast_to(linear_bias.astype(jnp.float32), (_LANES,))
    t = jnp.arange(_TOK, dtype=jnp.int32)
    tidx = (t % _LANES) * _L + t // _LANES
    p = _project(embd_weight, linear_weight.astype(jnp.float32))
    return _pool(p, ids_g, att_g, bias16, tidx)


# speedup vs baseline: 1.2150x; 1.2150x over previous
import jax
import jax.numpy as jnp
from jax import lax
from jax.experimental import pallas as pl
from jax.experimental.pallas import tpu as pltpu
from jax.experimental.pallas import tpu_sc as plsc

_B = 16384
_L = 200
_VOCAB = 1000000
_DIM = 32

_NC = 2
_NS = 16
_LANES = 16
_NW = _NC * _NS
_GRP = _B // _LANES
_GPW = _GRP // _NW
_TOK = _L * _LANES
_CH = 128
_NCH = _TOK // _CH

_VB = 4096
_PGRID = -(-_VOCAB // _VB)
_PLEN = _PGRID * _VB


def _proj_body(w_ref, tbl_ref, out_ref):
    r = lax.dot_general(w_ref[...], tbl_ref[...], (((1,), (1,)), ((), ())),
                        preferred_element_type=jnp.float32)
    out_ref[...] = r.reshape(_VB)


def _project(embd_weight, linear_weight):
    return pl.pallas_call(
        _proj_body,
        grid=(_PGRID,),
        in_specs=[
            pl.BlockSpec((1, _DIM), lambda i: (0, 0)),
            pl.BlockSpec((_VB, _DIM), lambda i: (i, 0)),
        ],
        out_specs=pl.BlockSpec((_VB,), lambda i: (i,)),
        out_shape=jax.ShapeDtypeStruct((_PLEN,), jnp.float32),
    )(linear_weight, embd_weight)


def _pool_body(p_hbm, ids_hbm, att_hbm, bias_hbm, tidx_hbm, out_hbm,
               tidx_v, gidx_v, idxt_v, attt_v, val_v,
               bias_v, out_v, ish, ash, semt, semg):
    cid = lax.axis_index("c")
    sid = lax.axis_index("s")
    wid = sid * _NC + cid
    pltpu.sync_copy(bias_hbm, bias_v)
    pltpu.sync_copy(tidx_hbm, tidx_v)
    soff = (sid * _TOK).astype(jnp.int32)
    for k in range(_L):
        ds = pl.ds(k * _LANES, _LANES)
        gidx_v[ds] = tidx_v[ds] + soff

    def group_body(gl, carry):
        g = wid * _GPW + gl
        my_ish = ish.at[pl.ds(soff, _TOK)]
        my_ash = ash.at[pl.ds(soff, _TOK)]
        pltpu.sync_copy(ids_hbm.at[pl.ds(g * _TOK, _TOK)], my_ish)
        pltpu.sync_copy(att_hbm.at[pl.ds(g * _TOK, _TOK)], my_ash)
        tcopies = []
        for j in range(_NCH):
            ds = pl.ds(j * _CH, _CH)
            tcopies.append(pltpu.async_copy(
                ish.at[gidx_v.at[ds]], idxt_v.at[ds], semt))
            tcopies.append(pltpu.async_copy(
                ash.at[gidx_v.at[ds]], attt_v.at[ds], semt))
        for c in tcopies:
            c.wait()
        gcopies = [
            pltpu.async_copy(
                p_hbm.at[idxt_v.at[pl.ds(j * _CH, _CH)]],
                val_v.at[pl.ds(j * _CH, _CH)],
                semg,
            )
            for j in range(_NCH)
        ]
        for c in gcopies:
            c.wait()

        def tok_body(i, tc):
            acc, den = tc
            a = attt_v[pl.ds(i * _LANES, _LANES)]
            v = val_v[pl.ds(i * _LANES, _LANES)]
            return acc + v * a, den + a

        zero = jnp.zeros((_LANES,), jnp.float32)
        acc, den = lax.fori_loop(0, _L, tok_body, (zero, zero))
        logit = acc / den + bias_v[...]
        out_v[...] = 1.0 / (1.0 + jnp.exp(-logit))
        pltpu.sync_copy(out_v, out_hbm.at[pl.ds(g * _LANES, _LANES)])
        return carry

    lax.fori_loop(0, _GPW, group_body, 0)


def _pool(p, ids_g, att_g, bias16, tidx):
    mesh = plsc.VectorSubcoreMesh(
        core_axis_name="c", subcore_axis_name="s",
        num_cores=_NC, num_subcores=_NS,
    )
    return pl.kernel(
        _pool_body,
        out_type=jax.ShapeDtypeStruct((_B,), jnp.float32),
        mesh=mesh,
        scratch_types=[
            pltpu.VMEM((_TOK,), jnp.int32),
            pltpu.VMEM((_TOK,), jnp.int32),
            pltpu.VMEM((_TOK,), jnp.int32),
            pltpu.VMEM((_TOK,), jnp.float32),
            pltpu.VMEM((_TOK,), jnp.float32),
            pltpu.VMEM((_LANES,), jnp.float32),
            pltpu.VMEM((_LANES,), jnp.float32),
            pltpu.VMEM_SHARED((_NS * _TOK,), jnp.int32),
            pltpu.VMEM_SHARED((_NS * _TOK,), jnp.float32),
            pltpu.SemaphoreType.DMA,
            pltpu.SemaphoreType.DMA,
        ],
    )(p, ids_g, att_g, bias16, tidx)


def kernel(ids, att_ids, embd_weight, linear_weight, linear_bias):
    ids_g = ids.astype(jnp.int32).reshape(-1)
    att_g = att_ids.astype(jnp.float32).reshape(-1)
    bias16 = jnp.broadcast_to(linear_bias.astype(jnp.float32), (_LANES,))
    t = jnp.arange(_TOK, dtype=jnp.int32)
    tidx = (t % _LANES) * _L + t // _LANES
    p = _project(embd_weight, linear_weight.astype(jnp.float32))
    return _pool(p, ids_g, att_g, bias16, tidx)


# speedup vs baseline: 2.8030x; 2.3070x over previous
import jax
import jax.numpy as jnp
from jax import lax
from jax.experimental import pallas as pl
from jax.experimental.pallas import tpu as pltpu
from jax.experimental.pallas import tpu_sc as plsc

_B = 16384
_L = 200
_VOCAB = 1000000
_DIM = 32

_NC = 2
_NS = 16
_LANES = 16
_NW = _NC * _NS
_SPW = _B // _NW
_KCH = _SPW // _LANES
_TB = 8
_CHW = 128

_VB = 4096
_PGRID = -(-_VOCAB // _VB)
_PLEN = _PGRID * _VB


def _proj_body(w_ref, tblt_ref, out_ref):
    r = lax.dot_general(w_ref[...], tblt_ref[...], (((1,), (0,)), ((), ())),
                        preferred_element_type=jnp.float32)
    out_ref[...] = r.reshape(_VB)


def _project(embd_t, linear_weight):
    return pl.pallas_call(
        _proj_body,
        grid=(_PGRID,),
        in_specs=[
            pl.BlockSpec((1, _DIM), lambda i: (0, 0)),
            pl.BlockSpec((_DIM, _VB), lambda i: (0, i)),
        ],
        out_specs=pl.BlockSpec((_VB,), lambda i: (i,)),
        out_shape=jax.ShapeDtypeStruct((_PLEN,), jnp.float32),
    )(linear_weight, embd_t)


def _pool_body(p_hbm, idst_hbm, attt_hbm, bias_hbm, out_hbm,
               idb, atb, vlb, acc_v, den_v, bias_v, out_v, semg):
    cid = lax.axis_index("c")
    sid = lax.axis_index("s")
    wid = sid * _NC + cid
    base = wid * _SPW
    zero = jnp.zeros((_LANES,), jnp.float32)
    for k in range(_KCH):
        ds = pl.ds(k * _LANES, _LANES)
        acc_v[ds] = zero
        den_v[ds] = zero

    def blk_body(bi, carry):
        pltpu.sync_copy(
            idst_hbm.at[pl.ds(bi * _TB, _TB), pl.ds(base, _SPW)], idb)
        pltpu.sync_copy(
            attt_hbm.at[pl.ds(bi * _TB, _TB), pl.ds(base, _SPW)], atb)
        cps = []
        for r in range(_TB):
            for j in range(_SPW // _CHW):
                ds = pl.ds(j * _CHW, _CHW)
                cps.append(pltpu.async_copy(
                    p_hbm.at[idb.at[r, ds]], vlb.at[r, ds], semg))
        for c in cps:
            c.wait()
        for r in range(_TB):
            for k in range(_KCH):
                ds = pl.ds(k * _LANES, _LANES)
                a = atb[r, ds]
                acc_v[ds] = acc_v[ds] + vlb[r, ds] * a
                den_v[ds] = den_v[ds] + a
        return carry

    lax.fori_loop(0, _L // _TB, blk_body, 0)

    pltpu.sync_copy(bias_hbm, bias_v)
    for k in range(_KCH):
        ds = pl.ds(k * _LANES, _LANES)
        logit = acc_v[ds] / den_v[ds] + bias_v[...]
        out_v[ds] = 1.0 / (1.0 + jnp.exp(-logit))
    pltpu.sync_copy(out_v, out_hbm.at[pl.ds(base, _SPW)])


def _pool(p, ids_t, att_t, bias16):
    mesh = plsc.VectorSubcoreMesh(
        core_axis_name="c", subcore_axis_name="s",
        num_cores=_NC, num_subcores=_NS,
    )
    return pl.kernel(
        _pool_body,
        out_type=jax.ShapeDtypeStruct((_B,), jnp.float32),
        mesh=mesh,
        scratch_types=[
            pltpu.VMEM((_TB, _SPW), jnp.int32),
            pltpu.VMEM((_TB, _SPW), jnp.float32),
            pltpu.VMEM((_TB, _SPW), jnp.float32),
            pltpu.VMEM((_SPW,), jnp.float32),
            pltpu.VMEM((_SPW,), jnp.float32),
            pltpu.VMEM((_LANES,), jnp.float32),
            pltpu.VMEM((_SPW,), jnp.float32),
            pltpu.SemaphoreType.DMA,
        ],
    )(p, ids_t, att_t, bias16)


def kernel(ids, att_ids, embd_weight, linear_weight, linear_bias):
    ids_t = ids.astype(jnp.int32).T
    att_t = att_ids.astype(jnp.float32).T
    embd_t = embd_weight.T
    bias16 = jnp.broadcast_to(linear_bias.astype(jnp.float32), (_LANES,))
    p = _project(embd_t, linear_weight.astype(jnp.float32))
    return _pool(p, ids_t, att_t, bias16)


# speedup vs baseline: 4.3517x; 1.5525x over previous
import jax
import jax.numpy as jnp
from jax import lax
from jax.experimental import pallas as pl
from jax.experimental.pallas import tpu as pltpu
from jax.experimental.pallas import tpu_sc as plsc

_B = 16384
_L = 200
_VOCAB = 1000000
_DIM = 32

_NC = 2
_NS = 16
_LANES = 16
_NW = _NC * _NS
_SPW = _B // _NW
_KCH = _SPW // _LANES
_TB = 8
_CHW = 128

_VB = 8192
_PGRID = -(-_VOCAB // _VB)
_PLEN = _PGRID * _VB


def _proj_body(w_ref, tblt_ref, out_ref):
    r = lax.dot_general(w_ref[...], tblt_ref[...], (((1,), (0,)), ((), ())),
                        preferred_element_type=jnp.float32)
    out_ref[...] = r.reshape(_VB)


def _project(embd_t, linear_weight):
    return pl.pallas_call(
        _proj_body,
        grid=(_PGRID,),
        in_specs=[
            pl.BlockSpec((1, _DIM), lambda i: (0, 0)),
            pl.BlockSpec((_DIM, _VB), lambda i: (0, i)),
        ],
        out_specs=pl.BlockSpec((_VB,), lambda i: (i,)),
        out_shape=jax.ShapeDtypeStruct((_PLEN,), jnp.float32),
    )(linear_weight, embd_t)


def _pool_body(p_hbm, idst_hbm, attt_hbm, bias_hbm, out_hbm,
               idb0, atb0, vlb0, idb1, atb1, vlb1,
               acc_v, den_v, bias_v, out_v,
               seml0, seml1, semg0, semg1):
    cid = lax.axis_index("c")
    sid = lax.axis_index("s")
    wid = sid * _NC + cid
    base = wid * _SPW
    bufs = ((idb0, atb0, vlb0, seml0, semg0),
            (idb1, atb1, vlb1, seml1, semg1))
    zero = jnp.zeros((_LANES,), jnp.float32)
    for k in range(_KCH):
        ds = pl.ds(k * _LANES, _LANES)
        acc_v[ds] = zero
        den_v[ds] = zero

    def load(bi, s):
        idb, atb, _, seml, _ = bufs[s]
        pltpu.async_copy(
            idst_hbm.at[pl.ds(bi * _TB, _TB), pl.ds(base, _SPW)], idb, seml)
        pltpu.async_copy(
            attt_hbm.at[pl.ds(bi * _TB, _TB), pl.ds(base, _SPW)], atb, seml)

    def fire(s):
        idb, atb, vlb, seml, semg = bufs[s]
        pltpu.make_async_copy(idst_hbm.at[pl.ds(0, _TB), pl.ds(0, _SPW)],
                              idb, seml).wait()
        pltpu.make_async_copy(attt_hbm.at[pl.ds(0, _TB), pl.ds(0, _SPW)],
                              atb, seml).wait()
        for r in range(_TB):
            for j in range(_SPW // _CHW):
                ds = pl.ds(j * _CHW, _CHW)
                pltpu.async_copy(p_hbm.at[idb.at[r, ds]], vlb.at[r, ds], semg)

    def drain_compute(s):
        idb, atb, vlb, _, semg = bufs[s]
        for r in range(_TB):
            for j in range(_SPW // _CHW):
                ds = pl.ds(j * _CHW, _CHW)
                pltpu.make_async_copy(p_hbm.at[idb.at[r, ds]],
                                      vlb.at[r, ds], semg).wait()
        for k in range(_KCH):
            ds = pl.ds(k * _LANES, _LANES)
            acc = acc_v[ds]
            den = den_v[ds]
            for r in range(_TB):
                a = atb[r, ds]
                acc = acc + vlb[r, ds] * a
                den = den + a
            acc_v[ds] = acc
            den_v[ds] = den

    load(0, 0)
    fire(0)
    load(1, 1)

    def pair_body(t, carry):
        fire(1)
        drain_compute(0)
        load(2 * t + 2, 0)
        fire(0)
        drain_compute(1)
        load(2 * t + 3, 1)
        return carry

    lax.fori_loop(0, (_L // _TB) // 2 - 1, pair_body, 0)
    fire(1)
    drain_compute(0)
    load(24, 0)
    fire(0)
    drain_compute(1)
    drain_compute(0)

    pltpu.sync_copy(bias_hbm, bias_v)
    for k in range(_KCH):
        ds = pl.ds(k * _LANES, _LANES)
        logit = acc_v[ds] / den_v[ds] + bias_v[...]
        out_v[ds] = 1.0 / (1.0 + jnp.exp(-logit))
    pltpu.sync_copy(out_v, out_hbm.at[pl.ds(base, _SPW)])


def _pool(p, ids_t, att_t, bias16):
    mesh = plsc.VectorSubcoreMesh(
        core_axis_name="c", subcore_axis_name="s",
        num_cores=_NC, num_subcores=_NS,
    )
    return pl.kernel(
        _pool_body,
        out_type=jax.ShapeDtypeStruct((_B,), jnp.float32),
        mesh=mesh,
        scratch_types=[
            pltpu.VMEM((_TB, _SPW), jnp.int32),
            pltpu.VMEM((_TB, _SPW), jnp.float32),
            pltpu.VMEM((_TB, _SPW), jnp.float32),
            pltpu.VMEM((_TB, _SPW), jnp.int32),
            pltpu.VMEM((_TB, _SPW), jnp.float32),
            pltpu.VMEM((_TB, _SPW), jnp.float32),
            pltpu.VMEM((_SPW,), jnp.float32),
            pltpu.VMEM((_SPW,), jnp.float32),
            pltpu.VMEM((_LANES,), jnp.float32),
            pltpu.VMEM((_SPW,), jnp.float32),
            pltpu.SemaphoreType.DMA,
            pltpu.SemaphoreType.DMA,
            pltpu.SemaphoreType.DMA,
            pltpu.SemaphoreType.DMA,
        ],
    )(p, ids_t, att_t, bias16)


def kernel(ids, att_ids, embd_weight, linear_weight, linear_bias):
    ids_t = ids.astype(jnp.int32).T
    att_t = att_ids.astype(jnp.float32).T
    embd_t = embd_weight.T
    bias16 = jnp.broadcast_to(linear_bias.astype(jnp.float32), (_LANES,))
    p = _project(embd_t, linear_weight.astype(jnp.float32))
    return _pool(p, ids_t, att_t, bias16)


# speedup vs baseline: 4.9797x; 1.1443x over previous
import jax
import jax.numpy as jnp
from jax import lax
from jax.experimental import pallas as pl
from jax.experimental.pallas import tpu as pltpu
from jax.experimental.pallas import tpu_sc as plsc

_B = 16384
_L = 200
_VOCAB = 1000000
_DIM = 32

_NC = 2
_NS = 16
_LANES = 16
_NW = _NC * _NS
_SPW = _B // _NW
_KCH = _SPW // _LANES
_TB = 8
_CHW = 128

_VB = 16384
_PGRID = -(-_VOCAB // _VB)
_PLEN = _PGRID * _VB


def _proj_body(w_ref, tblt_ref, out_ref):
    r = lax.dot_general(w_ref[...], tblt_ref[...], (((1,), (0,)), ((), ())),
                        preferred_element_type=jnp.float32)
    out_ref[...] = r.reshape(_VB)


def _project(embd_t, linear_weight):
    return pl.pallas_call(
        _proj_body,
        grid=(_PGRID,),
        in_specs=[
            pl.BlockSpec((1, _DIM), lambda i: (0, 0)),
            pl.BlockSpec((_DIM, _VB), lambda i: (0, i)),
        ],
        out_specs=pl.BlockSpec((_VB,), lambda i: (i,)),
        out_shape=jax.ShapeDtypeStruct((_PLEN,), jnp.float32),
    )(linear_weight, embd_t)


def _pool_body(p_hbm, idst_hbm, attt_hbm, bias_hbm, out_hbm,
               idb0, atb0, vlb0, idb1, atb1, vlb1,
               acc_v, den_v, bias_v, out_v,
               seml0, seml1, semg0, semg1):
    cid = lax.axis_index("c")
    sid = lax.axis_index("s")
    wid = sid * _NC + cid
    base = wid * _SPW
    bufs = ((idb0, atb0, vlb0, seml0, semg0),
            (idb1, atb1, vlb1, seml1, semg1))
    zero = jnp.zeros((_LANES,), jnp.float32)
    for k in range(_KCH):
        ds = pl.ds(k * _LANES, _LANES)
        acc_v[ds] = zero
        den_v[ds] = zero

    def load(bi, s):
        idb, atb, _, seml, _ = bufs[s]
        pltpu.async_copy(
            idst_hbm.at[pl.ds(bi * _TB, _TB), pl.ds(base, _SPW)], idb, seml)
        pltpu.async_copy(
            attt_hbm.at[pl.ds(bi * _TB, _TB), pl.ds(base, _SPW)], atb, seml)

    def fire(s):
        idb, atb, vlb, seml, semg = bufs[s]
        pltpu.make_async_copy(idst_hbm.at[pl.ds(0, _TB), pl.ds(0, _SPW)],
                              idb, seml).wait()
        pltpu.make_async_copy(attt_hbm.at[pl.ds(0, _TB), pl.ds(0, _SPW)],
                              atb, seml).wait()
        for r in range(_TB):
            for j in range(_SPW // _CHW):
                ds = pl.ds(j * _CHW, _CHW)
                pltpu.async_copy(p_hbm.at[idb.at[r, ds]], vlb.at[r, ds], semg)

    def drain_compute(s):
        idb, atb, vlb, _, semg = bufs[s]
        for r in range(_TB):
            for j in range(_SPW // _CHW):
                ds = pl.ds(j * _CHW, _CHW)
                pltpu.make_async_copy(p_hbm.at[idb.at[r, ds]],
                                      vlb.at[r, ds], semg).wait()
        for k in range(_KCH):
            ds = pl.ds(k * _LANES, _LANES)
            acc = acc_v[ds]
            den = den_v[ds]
            for r in range(_TB):
                a = atb[r, ds]
                acc = acc + vlb[r, ds] * a
                den = den + a
            acc_v[ds] = acc
            den_v[ds] = den

    load(0, 0)
    fire(0)
    load(1, 1)

    def pair_body(t, carry):
        fire(1)
        drain_compute(0)
        load(2 * t + 2, 0)
        fire(0)
        drain_compute(1)
        load(2 * t + 3, 1)
        return carry

    lax.fori_loop(0, (_L // _TB) // 2 - 1, pair_body, 0)
    fire(1)
    drain_compute(0)
    load(24, 0)
    fire(0)
    drain_compute(1)
    drain_compute(0)

    pltpu.sync_copy(bias_hbm, bias_v)
    for k in range(_KCH):
        ds = pl.ds(k * _LANES, _LANES)
        logit = acc_v[ds] / den_v[ds] + bias_v[...]
        out_v[ds] = 1.0 / (1.0 + jnp.exp(-logit))
    pltpu.sync_copy(out_v, out_hbm.at[pl.ds(base, _SPW)])


def _pool(p, ids_t, att_t, bias16):
    mesh = plsc.VectorSubcoreMesh(
        core_axis_name="c", subcore_axis_name="s",
        num_cores=_NC, num_subcores=_NS,
    )
    return pl.kernel(
        _pool_body,
        out_type=jax.ShapeDtypeStruct((_B,), jnp.float32),
        mesh=mesh,
        scratch_types=[
            pltpu.VMEM((_TB, _SPW), jnp.int32),
            pltpu.VMEM((_TB, _SPW), jnp.float32),
            pltpu.VMEM((_TB, _SPW), jnp.float32),
            pltpu.VMEM((_TB, _SPW), jnp.int32),
            pltpu.VMEM((_TB, _SPW), jnp.float32),
            pltpu.VMEM((_TB, _SPW), jnp.float32),
            pltpu.VMEM((_SPW,), jnp.float32),
            pltpu.VMEM((_SPW,), jnp.float32),
            pltpu.VMEM((_LANES,), jnp.float32),
            pltpu.VMEM((_SPW,), jnp.float32),
            pltpu.SemaphoreType.DMA,
            pltpu.SemaphoreType.DMA,
            pltpu.SemaphoreType.DMA,
            pltpu.SemaphoreType.DMA,
        ],
    )(p, ids_t, att_t, bias16)


def kernel(ids, att_ids, embd_weight, linear_weight, linear_bias):
    ids_t = ids.astype(jnp.int32).T
    att_t = att_ids.astype(jnp.float32).T
    embd_t = embd_weight.T
    bias16 = jnp.broadcast_to(linear_bias.astype(jnp.float32), (_LANES,))
    p = _project(embd_t, linear_weight.astype(jnp.float32))
    return _pool(p, ids_t, att_t, bias16)


# speedup vs baseline: 5.4220x; 1.0888x over previous
import jax
import jax.numpy as jnp
from jax import lax
from jax.experimental import pallas as pl
from jax.experimental.pallas import tpu as pltpu
from jax.experimental.pallas import tpu_sc as plsc

_B = 16384
_L = 200
_VOCAB = 1000000
_DIM = 32

_NC = 2
_NS = 16
_LANES = 16
_NW = _NC * _NS
_SPW = _B // _NW
_KCH = _SPW // _LANES
_TB = 8
_CHW = 128

_VB = 32768
_PGRID = -(-_VOCAB // _VB)
_PLEN = _PGRID * _VB


def _proj_body(w_ref, tblt_ref, out_ref):
    r = lax.dot_general(w_ref[...], tblt_ref[...], (((1,), (0,)), ((), ())),
                        preferred_element_type=jnp.float32)
    out_ref[...] = r.reshape(_VB)


def _project(embd_t, linear_weight):
    return pl.pallas_call(
        _proj_body,
        grid=(_PGRID,),
        in_specs=[
            pl.BlockSpec((1, _DIM), lambda i: (0, 0)),
            pl.BlockSpec((_DIM, _VB), lambda i: (0, i)),
        ],
        out_specs=pl.BlockSpec((_VB,), lambda i: (i,)),
        out_shape=jax.ShapeDtypeStruct((_PLEN,), jnp.float32),
    )(linear_weight, embd_t)


def _pool_body(p_hbm, idst_hbm, attt_hbm, bias_hbm, out_hbm,
               idb0, atb0, vlb0, idb1, atb1, vlb1,
               acc_v, den_v, bias_v, out_v,
               seml0, seml1, semg0, semg1):
    cid = lax.axis_index("c")
    sid = lax.axis_index("s")
    wid = sid * _NC + cid
    base = wid * _SPW
    bufs = ((idb0, atb0, vlb0, seml0, semg0),
            (idb1, atb1, vlb1, seml1, semg1))
    zero = jnp.zeros((_LANES,), jnp.float32)
    for k in range(_KCH):
        ds = pl.ds(k * _LANES, _LANES)
        acc_v[ds] = zero
        den_v[ds] = zero

    def load(bi, s):
        idb, atb, _, seml, _ = bufs[s]
        pltpu.async_copy(
            idst_hbm.at[pl.ds(bi * _TB, _TB), pl.ds(base, _SPW)], idb, seml)
        pltpu.async_copy(
            attt_hbm.at[pl.ds(bi * _TB, _TB), pl.ds(base, _SPW)], atb, seml)

    def fire(s):
        idb, atb, vlb, seml, semg = bufs[s]
        pltpu.make_async_copy(idst_hbm.at[pl.ds(0, _TB), pl.ds(0, _SPW)],
                              idb, seml).wait()
        pltpu.make_async_copy(attt_hbm.at[pl.ds(0, _TB), pl.ds(0, _SPW)],
                              atb, seml).wait()
        for r in range(_TB):
            for j in range(_SPW // _CHW):
                ds = pl.ds(j * _CHW, _CHW)
                pltpu.async_copy(p_hbm.at[idb.at[r, ds]], vlb.at[r, ds], semg)

    def drain_compute(s):
        idb, atb, vlb, _, semg = bufs[s]
        for r in range(_TB):
            for j in range(_SPW // _CHW):
                ds = pl.ds(j * _CHW, _CHW)
                pltpu.make_async_copy(p_hbm.at[idb.at[r, ds]],
                                      vlb.at[r, ds], semg).wait()
        for k in range(_KCH):
            ds = pl.ds(k * _LANES, _LANES)
            acc = acc_v[ds]
            den = den_v[ds]
            for r in range(_TB):
                a = atb[r, ds]
                acc = acc + vlb[r, ds] * a
                den = den + a
            acc_v[ds] = acc
            den_v[ds] = den

    load(0, 0)
    fire(0)
    load(1, 1)

    def pair_body(t, carry):
        fire(1)
        drain_compute(0)
        load(2 * t + 2, 0)
        fire(0)
        drain_compute(1)
        load(2 * t + 3, 1)
        return carry

    lax.fori_loop(0, (_L // _TB) // 2 - 1, pair_body, 0)
    fire(1)
    drain_compute(0)
    load(24, 0)
    fire(0)
    drain_compute(1)
    drain_compute(0)

    pltpu.sync_copy(bias_hbm, bias_v)
    for k in range(_KCH):
        ds = pl.ds(k * _LANES, _LANES)
        logit = acc_v[ds] / den_v[ds] + bias_v[...]
        out_v[ds] = 1.0 / (1.0 + jnp.exp(-logit))
    pltpu.sync_copy(out_v, out_hbm.at[pl.ds(base, _SPW)])


def _pool(p, ids_t, att_t, bias16):
    mesh = plsc.VectorSubcoreMesh(
        core_axis_name="c", subcore_axis_name="s",
        num_cores=_NC, num_subcores=_NS,
    )
    return pl.kernel(
        _pool_body,
        out_type=jax.ShapeDtypeStruct((_B,), jnp.float32),
        mesh=mesh,
        scratch_types=[
            pltpu.VMEM((_TB, _SPW), jnp.int32),
            pltpu.VMEM((_TB, _SPW), jnp.float32),
            pltpu.VMEM((_TB, _SPW), jnp.float32),
            pltpu.VMEM((_TB, _SPW), jnp.int32),
            pltpu.VMEM((_TB, _SPW), jnp.float32),
            pltpu.VMEM((_TB, _SPW), jnp.float32),
            pltpu.VMEM((_SPW,), jnp.float32),
            pltpu.VMEM((_SPW,), jnp.float32),
            pltpu.VMEM((_LANES,), jnp.float32),
            pltpu.VMEM((_SPW,), jnp.float32),
            pltpu.SemaphoreType.DMA,
            pltpu.SemaphoreType.DMA,
            pltpu.SemaphoreType.DMA,
            pltpu.SemaphoreType.DMA,
        ],
    )(p, ids_t, att_t, bias16)


def kernel(ids, att_ids, embd_weight, linear_weight, linear_bias):
    ids_t = ids.astype(jnp.int32).T
    att_t = att_ids.astype(jnp.float32).T
    embd_t = embd_weight.T
    bias16 = jnp.broadcast_to(linear_bias.astype(jnp.float32), (_LANES,))
    p = _project(embd_t, linear_weight.astype(jnp.float32))
    return _pool(p, ids_t, att_t, bias16)


# speedup vs baseline: 5.5352x; 1.0209x over previous
import jax
import jax.numpy as jnp
from jax import lax
from jax.experimental import pallas as pl
from jax.experimental.pallas import tpu as pltpu
from jax.experimental.pallas import tpu_sc as plsc

_B = 16384
_L = 200
_VOCAB = 1000000
_DIM = 32

_NC = 2
_NS = 16
_LANES = 16
_NW = _NC * _NS
_SPW = _B // _NW
_KCH = _SPW // _LANES
_TB = 8
_CHW = 128

_VB = 65536
_PGRID = -(-_VOCAB // _VB)
_PLEN = _PGRID * _VB


def _proj_body(w_ref, tblt_ref, out_ref):
    r = lax.dot_general(w_ref[...], tblt_ref[...], (((1,), (0,)), ((), ())),
                        preferred_element_type=jnp.float32)
    out_ref[...] = r.reshape(_VB)


def _project(embd_t, linear_weight):
    return pl.pallas_call(
        _proj_body,
        grid=(_PGRID,),
        in_specs=[
            pl.BlockSpec((1, _DIM), lambda i: (0, 0)),
            pl.BlockSpec((_DIM, _VB), lambda i: (0, i)),
        ],
        out_specs=pl.BlockSpec((_VB,), lambda i: (i,)),
        out_shape=jax.ShapeDtypeStruct((_PLEN,), jnp.float32),
    )(linear_weight, embd_t)


def _pool_body(p_hbm, idst_hbm, attt_hbm, bias_hbm, out_hbm,
               idb0, atb0, vlb0, idb1, atb1, vlb1,
               acc_v, den_v, bias_v, out_v,
               seml0, seml1, semg0, semg1):
    cid = lax.axis_index("c")
    sid = lax.axis_index("s")
    wid = sid * _NC + cid
    base = wid * _SPW
    bufs = ((idb0, atb0, vlb0, seml0, semg0),
            (idb1, atb1, vlb1, seml1, semg1))
    zero = jnp.zeros((_LANES,), jnp.float32)
    for k in range(_KCH):
        ds = pl.ds(k * _LANES, _LANES)
        acc_v[ds] = zero
        den_v[ds] = zero

    def load(bi, s):
        idb, atb, _, seml, _ = bufs[s]
        pltpu.async_copy(
            idst_hbm.at[pl.ds(bi * _TB, _TB), pl.ds(base, _SPW)], idb, seml)
        pltpu.async_copy(
            attt_hbm.at[pl.ds(bi * _TB, _TB), pl.ds(base, _SPW)], atb, seml)

    def fire(s):
        idb, atb, vlb, seml, semg = bufs[s]
        pltpu.make_async_copy(idst_hbm.at[pl.ds(0, _TB), pl.ds(0, _SPW)],
                              idb, seml).wait()
        pltpu.make_async_copy(attt_hbm.at[pl.ds(0, _TB), pl.ds(0, _SPW)],
                              atb, seml).wait()
        for r in range(_TB):
            for j in range(_SPW // _CHW):
                ds = pl.ds(j * _CHW, _CHW)
                pltpu.async_copy(p_hbm.at[idb.at[r, ds]], vlb.at[r, ds], semg)

    def drain_compute(s):
        idb, atb, vlb, _, semg = bufs[s]
        for r in range(_TB):
            for j in range(_SPW // _CHW):
                ds = pl.ds(j * _CHW, _CHW)
                pltpu.make_async_copy(p_hbm.at[idb.at[r, ds]],
                                      vlb.at[r, ds], semg).wait()
        for k in range(_KCH):
            ds = pl.ds(k * _LANES, _LANES)
            acc = acc_v[ds]
            den = den_v[ds]
            for r in range(_TB):
                a = atb[r, ds]
                acc = acc + vlb[r, ds] * a
                den = den + a
            acc_v[ds] = acc
            den_v[ds] = den

    load(0, 0)
    fire(0)
    load(1, 1)

    def pair_body(t, carry):
        fire(1)
        drain_compute(0)
        load(2 * t + 2, 0)
        fire(0)
        drain_compute(1)
        load(2 * t + 3, 1)
        return carry

    lax.fori_loop(0, (_L // _TB) // 2 - 1, pair_body, 0)
    fire(1)
    drain_compute(0)
    load(24, 0)
    fire(0)
    drain_compute(1)
    drain_compute(0)

    pltpu.sync_copy(bias_hbm, bias_v)
    for k in range(_KCH):
        ds = pl.ds(k * _LANES, _LANES)
        logit = acc_v[ds] / den_v[ds] + bias_v[...]
        out_v[ds] = 1.0 / (1.0 + jnp.exp(-logit))
    pltpu.sync_copy(out_v, out_hbm.at[pl.ds(base, _SPW)])


def _pool(p, ids_t, att_t, bias16):
    mesh = plsc.VectorSubcoreMesh(
        core_axis_name="c", subcore_axis_name="s",
        num_cores=_NC, num_subcores=_NS,
    )
    return pl.kernel(
        _pool_body,
        out_type=jax.ShapeDtypeStruct((_B,), jnp.float32),
        mesh=mesh,
        scratch_types=[
            pltpu.VMEM((_TB, _SPW), jnp.int32),
            pltpu.VMEM((_TB, _SPW), jnp.float32),
            pltpu.VMEM((_TB, _SPW), jnp.float32),
            pltpu.VMEM((_TB, _SPW), jnp.int32),
            pltpu.VMEM((_TB, _SPW), jnp.float32),
            pltpu.VMEM((_TB, _SPW), jnp.float32),
            pltpu.VMEM((_SPW,), jnp.float32),
            pltpu.VMEM((_SPW,), jnp.float32),
            pltpu.VMEM((_LANES,), jnp.float32),
            pltpu.VMEM((_SPW,), jnp.float32),
            pltpu.SemaphoreType.DMA,
            pltpu.SemaphoreType.DMA,
            pltpu.SemaphoreType.DMA,
            pltpu.SemaphoreType.DMA,
        ],
    )(p, ids_t, att_t, bias16)


def kernel(ids, att_ids, embd_weight, linear_weight, linear_bias):
    ids_t = ids.astype(jnp.int32).T
    att_t = att_ids.astype(jnp.float32).T
    embd_t = embd_weight.T
    bias16 = jnp.broadcast_to(linear_bias.astype(jnp.float32), (_LANES,))
    p = _project(embd_t, linear_weight.astype(jnp.float32))
    return _pool(p, ids_t, att_t, bias16)
